# speedup vs baseline: 1.6967x; 1.6967x over previous
import functools

import jax
import jax.numpy as jnp
from jax import lax
from jax.experimental import pallas as pl
from jax.experimental.pallas import tpu as pltpu
from jax.experimental.pallas import tpu_sc as plsc

VOCAB = 100000
D_MODEL = 4096
BATCH = 4
SEQ = 4096

N_IDX = BATCH * SEQ
NUM_WORKERS = 32
PER_W = N_IDX // NUM_WORKERS
ROWS = 16
CHUNKS = PER_W // ROWS


@functools.partial(
    pl.kernel,
    mesh=plsc.VectorSubcoreMesh(core_axis_name="c", subcore_axis_name="s"),
    out_type=jax.ShapeDtypeStruct((N_IDX, D_MODEL), jnp.float32),
    scratch_types=[
        pltpu.VMEM((PER_W,), jnp.int32),
        pltpu.VMEM((ROWS, D_MODEL), jnp.float32),
        pltpu.SemaphoreType.DMA,
    ],
)
def _embed_gather(ids_hbm, table_hbm, out_hbm, idx_v, buf, gsem):
    wid = lax.axis_index("s") * 2 + lax.axis_index("c")
    base = wid * PER_W
    pltpu.sync_copy(ids_hbm.at[pl.ds(base, PER_W)], idx_v)

    def body(c, carry):
        pltpu.async_copy(
            table_hbm.at[idx_v.at[pl.ds(c * ROWS, ROWS)]], buf, gsem
        ).wait()
        pltpu.sync_copy(buf, out_hbm.at[pl.ds(base + c * ROWS, ROWS)])
        return carry

    lax.fori_loop(0, CHUNKS, body, 0)


def kernel(input_ids, table):
    ids = input_ids.reshape(N_IDX).astype(jnp.int32)
    out = _embed_gather(ids, table)
    return out.reshape(BATCH, SEQ, D_MODEL)


# speedup vs baseline: 1.8331x; 1.0804x over previous
import functools

import jax
import jax.numpy as jnp
from jax import lax
from jax.experimental import pallas as pl
from jax.experimental.pallas import tpu as pltpu
from jax.experimental.pallas import tpu_sc as plsc

VOCAB = 100000
D_MODEL = 4096
BATCH = 4
SEQ = 4096

N_IDX = BATCH * SEQ
NUM_WORKERS = 32
PER_W = N_IDX // NUM_WORKERS
ROWS = 8
CHUNKS = PER_W // ROWS
NBUF = 2


@functools.partial(
    pl.kernel,
    mesh=plsc.VectorSubcoreMesh(core_axis_name="c", subcore_axis_name="s"),
    out_type=jax.ShapeDtypeStruct((N_IDX, D_MODEL), jnp.float32),
    scratch_types=[
        pltpu.VMEM((PER_W,), jnp.int32),
        pltpu.VMEM((NBUF, ROWS, D_MODEL), jnp.float32),
        pltpu.SemaphoreType.DMA,
        pltpu.SemaphoreType.DMA,
        pltpu.SemaphoreType.DMA,
        pltpu.SemaphoreType.DMA,
    ],
)
def _embed_gather(ids_hbm, table_hbm, out_hbm, idx_v, buf, g0, g1, s0, s1):
    gsem = (g0, g1)
    ssem = (s0, s1)
    wid = lax.axis_index("s") * 2 + lax.axis_index("c")
    base = wid * PER_W
    pltpu.sync_copy(ids_hbm.at[pl.ds(base, PER_W)], idx_v)

    def start_gather(c, b):
        pltpu.async_copy(
            table_hbm.at[idx_v.at[pl.ds(c * ROWS, ROWS)]], buf.at[b], gsem[b]
        )

    def wait_gather(b):
        pltpu.make_async_copy(
            table_hbm.at[pl.ds(0, ROWS)], buf.at[b], gsem[b]
        ).wait()

    def start_scatter(c, b):
        pltpu.async_copy(
            buf.at[b], out_hbm.at[pl.ds(base + c * ROWS, ROWS)], ssem[b]
        )

    def wait_scatter(b):
        pltpu.make_async_copy(
            table_hbm.at[pl.ds(0, ROWS)], buf.at[b], ssem[b]
        ).wait()

    for b in range(NBUF):
        start_gather(b, b)

    def outer(i, carry):
        for b in range(NBUF):
            c = i * NBUF + b
            wait_gather(b)
            start_scatter(c, b)
            wait_scatter(b)
            start_gather(c + NBUF, b)
        return carry

    lax.fori_loop(0, (CHUNKS - NBUF) // NBUF, outer, 0)

    for b in range(NBUF):
        c = CHUNKS - NBUF + b
        wait_gather(b)
        start_scatter(c, b)
    for b in range(NBUF):
        wait_scatter(b)


def kernel(input_ids, table):
    ids = input_ids.reshape(N_IDX).astype(jnp.int32)
    out = _embed_gather(ids, table)
    return out.reshape(BATCH, SEQ, D_MODEL)
